# Initial kernel scaffold; baseline (speedup 1.0000x reference)
import jax
import jax.numpy as jnp
from jax.experimental import pallas as pl


def kernel(feat, class_map, W):
    raise NotImplementedError("write your pallas kernel here")


# speedup vs baseline: 1.2088x; 1.2088x over previous
import jax
import jax.numpy as jnp
from jax import lax
from jax.experimental import pallas as pl


def _body(cm_ref, feat_ref, wt_ref, out_ref):
    feat = feat_ref[0]
    wt = wt_ref[...]
    cm = cm_ref[0]
    l, c = feat.shape
    ks = wt.shape[1]
    s = ks // 8

    conf = jnp.dot(feat, wt, preferred_element_type=jnp.float32)
    kcol = lax.broadcasted_iota(jnp.int32, (l, ks), 1) // s
    mask = cm == kcol
    x = jnp.where(mask, conf, -1e30)
    m = jnp.max(x, axis=0, keepdims=True)
    e = jnp.where(mask, jnp.exp(x - m), 0.0)
    denom = jnp.sum(e, axis=0, keepdims=True)
    wts = e / jnp.maximum(denom, 1e-30)
    out = lax.dot_general(wts, feat, (((0,), (0,)), ((), ())),
                          preferred_element_type=jnp.float32)
    out_ref[0] = out


def kernel(feat, class_map, W):
    n, l, c = feat.shape
    k, s = W.shape[0], W.shape[1]
    wt = W.reshape(k * s, c).T
    cm3 = class_map.reshape(n, l, 1)
    return pl.pallas_call(
        _body,
        grid=(n,),
        in_specs=[
            pl.BlockSpec((1, l, 1), lambda i: (i, 0, 0)),
            pl.BlockSpec((1, l, c), lambda i: (i, 0, 0)),
            pl.BlockSpec((c, k * s), lambda i: (0, 0)),
        ],
        out_specs=pl.BlockSpec((1, k * s, c), lambda i: (i, 0, 0)),
        out_shape=jax.ShapeDtypeStruct((n, k * s, c), jnp.float32),
    )(cm3, feat, wt)
